# baseline (speedup 1.0000x reference)
import dataclasses
import functools

import jax
import jax.numpy as jnp
from jax import lax
from jax.experimental import pallas as pl
from jax.experimental.pallas import tpu as pltpu
from jax.experimental.pallas import tpu_sc as plsc

_L = 200
_E = 32
_LANES = 16
_W = 128
_NB = 4
_NWORK = 32
_TS = 129


def _compiler_params():
    cp = pltpu.CompilerParams(use_tc_tiling_on_sc=False)
    if "needs_layout_passes" in pltpu.CompilerParams.__dataclass_fields__:
        cp = dataclasses.replace(cp, needs_layout_passes=False)
    return cp


def _sc_transpose_table(tt):
    V = tt.shape[1]
    ngrp = V // _W
    gpw = ngrp // _NWORK
    rem = ngrp - gpw * _NWORK
    mesh = plsc.VectorSubcoreMesh(core_axis_name="core",
                                  subcore_axis_name="subcore")

    @functools.partial(
        pl.kernel,
        out_type=jax.ShapeDtypeStruct((V * _E // _W, _W), jnp.float32),
        mesh=mesh,
        compiler_params=_compiler_params(),
        scratch_types=(
            [pltpu.VMEM((_NB, _E, _TS), jnp.float32),
             pltpu.VMEM((_NB, _E, _W), jnp.float32)]
            + [pltpu.SemaphoreType.DMA] * (2 * _NB)
        ),
    )
    def ka(tt_hbm, out_hbm, ibufs, obufs, *sems):
        gsem = sems[:_NB]
        wsem = sems[_NB:]
        cid = lax.axis_index("core")
        sid = lax.axis_index("subcore")
        wid = sid * 2 + cid
        base_g = wid * gpw

        iota = lax.iota(jnp.int32, _LANES)
        row_h = [iota + 16 * h for h in range(2)]

        def g_start(g, b):
            pltpu.async_copy(
                tt_hbm.at[:, pl.ds((base_g + g) * _W, _W)],
                ibufs.at[b, :, pl.ds(0, _W)], gsem[b])

        def g_wait(g, b):
            pltpu.make_async_copy(
                tt_hbm.at[:, pl.ds((base_g + g) * _W, _W)],
                ibufs.at[b, :, pl.ds(0, _W)], gsem[b]).wait()

        def wb(g, b):
            return pltpu.make_async_copy(
                obufs.at[b],
                out_hbm.at[pl.ds((base_g + g) * _E, _E)], wsem[b])

        def transpose_group(b, nrow):
            @pl.loop(0, nrow)
            def _(r):
                for j in range(8):
                    col = jnp.full((_LANES,), 4 * r + j // 2, jnp.int32)
                    v = plsc.load_gather(ibufs.at[b], [row_h[j % 2], col])
                    obufs[b, r, pl.ds(16 * j, _LANES)] = v

        g_start(0, 0)

        @pl.loop(0, gpw, step=_NB)
        def _(g0):
            for b in range(_NB):
                g = g0 + b

                @pl.when(g + 1 < gpw)
                def _():
                    g_start(g + 1, (b + 1) % _NB)

                g_wait(g, b)

                @pl.when(g >= _NB)
                def _():
                    wb(g - _NB, b).wait()

                transpose_group(b, _E)
                wb(g, b).start()

        for b in range(_NB):
            wb(gpw - _NB + b, b).wait()

        @pl.when(wid == _NWORK - 1)
        def _():
            for k in range(rem):
                gg = _NWORK * gpw + k
                pltpu.sync_copy(tt_hbm.at[:, pl.ds(gg * _W, _W)],
                                ibufs.at[0, :, pl.ds(0, _W)])
                transpose_group(0, _E)
                pltpu.sync_copy(obufs.at[0],
                                out_hbm.at[pl.ds(gg * _E, _E)])
            tail = V - ngrp * _W
            pltpu.sync_copy(tt_hbm.at[:, pl.ds(ngrp * _W, tail)],
                            ibufs.at[0, :, pl.ds(0, tail)])
            transpose_group(0, tail * _E // _W)
            pltpu.sync_copy(obufs.at[0, pl.ds(0, tail * _E // _W)],
                            out_hbm.at[pl.ds(ngrp * _E, tail * _E // _W)])

    return ka(tt)


def _sc_embed(x2, token_table, pos_table):
    nwin = x2.shape[0]
    wpw = nwin // _NWORK
    nbatch = x2.shape[1] * 32
    mesh = plsc.VectorSubcoreMesh(core_axis_name="core",
                                  subcore_axis_name="subcore")

    @functools.partial(
        pl.kernel,
        out_type=jax.ShapeDtypeStruct((_L * _E, nbatch), jnp.float32),
        mesh=mesh,
        compiler_params=_compiler_params(),
        scratch_types=(
            [pltpu.VMEM((wpw, _W), jnp.int32),
             pltpu.VMEM((_L, _E), jnp.float32),
             pltpu.VMEM((_NB, _W, _E), jnp.float32),
             pltpu.VMEM((_NB, _E, _TS), jnp.float32)]
            + [pltpu.SemaphoreType.DMA] * (2 * _NB)
        ),
    )
    def k(x_hbm, tok_hbm, pos_hbm, out_hbm, idx_all, pos_v,
          bufs, tbufs, *sems):
        gsem = sems[:_NB]
        wsem = sems[_NB:2 * _NB]
        cid = lax.axis_index("core")
        sid = lax.axis_index("subcore")
        wid = sid * 2 + cid
        base_win = wid * wpw

        pltpu.sync_copy(x_hbm.at[pl.ds(base_win, wpw)], idx_all)
        pltpu.sync_copy(pos_hbm, pos_v)

        iota = lax.iota(jnp.int32, _LANES)
        e_h = [iota + 16 * h for h in range(2)]

        def gather_start(w, b):
            pltpu.async_copy(tok_hbm.at[idx_all.at[w]], bufs.at[b], gsem[b])

        def wb_copy(w_glob, b):
            l = w_glob >> 5
            bt = w_glob & 31
            return pltpu.make_async_copy(
                tbufs.at[b, :, pl.ds(0, _W)],
                out_hbm.at[pl.ds(l * _E, _E), pl.ds(bt * _W, _W)],
                wsem[b])

        gather_start(0, 0)

        @pl.loop(0, wpw, step=_NB)
        def _(g):
            for b in range(_NB):
                w = g + b
                wg = base_win + w

                @pl.when(w >= _NB)
                def _():
                    wb_copy(wg - _NB, b).wait()

                @pl.when(w + 1 < wpw)
                def _():
                    gather_start(w + 1, (b + 1) % _NB)

                pltpu.make_async_copy(tok_hbm.at[idx_all.at[w]],
                                      bufs.at[b], gsem[b]).wait()

                l = wg >> 5
                pv = [pos_v[l, pl.ds(16 * h, _LANES)] for h in range(2)]

                @pl.loop(0, _W, step=4)
                def _(t0):
                    for dt in range(4):
                        col = jnp.full((_LANES,), t0 + dt, jnp.int32)
                        for h in range(2):
                            v = bufs[b, t0 + dt, pl.ds(16 * h, _LANES)]
                            plsc.store_scatter(
                                tbufs.at[b], [e_h[h], col], v + pv[h])

                wb_copy(wg, b).start()

        for b in range(_NB):
            wb_copy(base_win + wpw - _NB + b, b).wait()

    return k(x2, token_table, pos_table)


def kernel(x, token_table, pos_table):
    Bseq, L = x.shape
    x2 = jnp.transpose(x).astype(jnp.int32).reshape(-1, _W)
    tt_rm = _sc_transpose_table(jnp.transpose(token_table))
    tt = tt_rm.reshape(token_table.shape)
    p2 = _sc_embed(x2, tt, pos_table)
    p3 = p2.reshape(L, _E, Bseq)
    return jnp.transpose(p3, (2, 0, 1))


# speedup vs baseline: 2.7942x; 2.7942x over previous
import dataclasses
import functools

import jax
import jax.numpy as jnp
from jax import lax
from jax.experimental import pallas as pl
from jax.experimental.pallas import tpu as pltpu
from jax.experimental.pallas import tpu_sc as plsc

_L = 200
_E = 32
_LANES = 16
_W = 128
_NB = 4
_NWORK = 32
_TS = 129


def _compiler_params(tc_tiling=False):
    cp = pltpu.CompilerParams(use_tc_tiling_on_sc=tc_tiling)
    if "needs_layout_passes" in pltpu.CompilerParams.__dataclass_fields__:
        cp = dataclasses.replace(cp, needs_layout_passes=False)
    return cp


def _sc_transpose_table(tt):
    V = tt.shape[1]
    ngrp = V // _W
    gpw = ngrp // _NWORK
    rem = ngrp - gpw * _NWORK
    mesh = plsc.VectorSubcoreMesh(core_axis_name="core",
                                  subcore_axis_name="subcore")

    @functools.partial(
        pl.kernel,
        out_type=jax.ShapeDtypeStruct((V * _E // _W, _W), jnp.float32),
        mesh=mesh,
        compiler_params=_compiler_params(tc_tiling=True),
        scratch_types=(
            [pltpu.VMEM((_NB, _E, _TS), jnp.float32),
             pltpu.VMEM((_NB, _E, _W), jnp.float32)]
            + [pltpu.SemaphoreType.DMA] * (2 * _NB)
        ),
    )
    def ka(tt_hbm, out_hbm, ibufs, obufs, *sems):
        gsem = sems[:_NB]
        wsem = sems[_NB:]
        cid = lax.axis_index("core")
        sid = lax.axis_index("subcore")
        wid = sid * 2 + cid
        base_g = wid * gpw

        iota = lax.iota(jnp.int32, _LANES)
        row_h = [iota + 16 * h for h in range(2)]

        def g_start(g, b):
            pltpu.async_copy(
                tt_hbm.at[:, pl.ds((base_g + g) * _W, _W)],
                ibufs.at[b, :, pl.ds(0, _W)], gsem[b])

        def g_wait(g, b):
            pltpu.make_async_copy(
                tt_hbm.at[:, pl.ds((base_g + g) * _W, _W)],
                ibufs.at[b, :, pl.ds(0, _W)], gsem[b]).wait()

        def wb(g, b):
            return pltpu.make_async_copy(
                obufs.at[b],
                out_hbm.at[pl.ds((base_g + g) * _E, _E)], wsem[b])

        def transpose_group(b, nrow):
            @pl.loop(0, nrow)
            def _(r):
                for j in range(8):
                    col = jnp.full((_LANES,), 4 * r + j // 2, jnp.int32)
                    v = plsc.load_gather(ibufs.at[b], [row_h[j % 2], col])
                    obufs[b, r, pl.ds(16 * j, _LANES)] = v

        g_start(0, 0)

        @pl.loop(0, gpw, step=_NB)
        def _(g0):
            for b in range(_NB):
                g = g0 + b

                @pl.when(g + 1 < gpw)
                def _():
                    g_start(g + 1, (b + 1) % _NB)

                g_wait(g, b)

                @pl.when(g >= _NB)
                def _():
                    wb(g - _NB, b).wait()

                transpose_group(b, _E)
                wb(g, b).start()

        for b in range(_NB):
            wb(gpw - _NB + b, b).wait()

        @pl.when(wid == _NWORK - 1)
        def _():
            for k in range(rem):
                gg = _NWORK * gpw + k
                pltpu.sync_copy(tt_hbm.at[:, pl.ds(gg * _W, _W)],
                                ibufs.at[0, :, pl.ds(0, _W)])
                transpose_group(0, _E)
                pltpu.sync_copy(obufs.at[0],
                                out_hbm.at[pl.ds(gg * _E, _E)])
            tail = V - ngrp * _W
            off = jnp.where(wid >= 0, ngrp * _W, 0)
            pltpu.sync_copy(tt_hbm.at[:, pl.ds(off, _W)],
                            ibufs.at[0, :, pl.ds(0, _W)])
            transpose_group(0, tail * _E // _W)
            pltpu.sync_copy(obufs.at[0, pl.ds(0, tail * _E // _W)],
                            out_hbm.at[pl.ds(ngrp * _E, tail * _E // _W)])

    return ka(tt)


def _sc_embed(x2, token_table, pos_table):
    nwin = x2.shape[0]
    wpw = nwin // _NWORK
    nbatch = x2.shape[1] * 32
    mesh = plsc.VectorSubcoreMesh(core_axis_name="core",
                                  subcore_axis_name="subcore")

    @functools.partial(
        pl.kernel,
        out_type=jax.ShapeDtypeStruct((_L * _E, nbatch), jnp.float32),
        mesh=mesh,
        compiler_params=_compiler_params(),
        scratch_types=(
            [pltpu.VMEM((wpw, _W), jnp.int32),
             pltpu.VMEM((_L, _E), jnp.float32),
             pltpu.VMEM((_NB, _W, _E), jnp.float32),
             pltpu.VMEM((_NB, _E, _TS), jnp.float32)]
            + [pltpu.SemaphoreType.DMA] * (2 * _NB)
        ),
    )
    def k(x_hbm, tok_hbm, pos_hbm, out_hbm, idx_all, pos_v,
          bufs, tbufs, *sems):
        gsem = sems[:_NB]
        wsem = sems[_NB:2 * _NB]
        cid = lax.axis_index("core")
        sid = lax.axis_index("subcore")
        wid = sid * 2 + cid
        base_win = wid * wpw

        pltpu.sync_copy(x_hbm.at[pl.ds(base_win, wpw)], idx_all)
        pltpu.sync_copy(pos_hbm, pos_v)

        iota = lax.iota(jnp.int32, _LANES)
        e_h = [iota + 16 * h for h in range(2)]

        def gather_start(w, b):
            pltpu.async_copy(tok_hbm.at[idx_all.at[w]], bufs.at[b], gsem[b])

        def wb_copy(w_glob, b):
            l = w_glob >> 5
            bt = w_glob & 31
            return pltpu.make_async_copy(
                tbufs.at[b, :, pl.ds(0, _W)],
                out_hbm.at[pl.ds(l * _E, _E), pl.ds(bt * _W, _W)],
                wsem[b])

        gather_start(0, 0)

        @pl.loop(0, wpw, step=_NB)
        def _(g):
            for b in range(_NB):
                w = g + b
                wg = base_win + w

                @pl.when(w >= _NB)
                def _():
                    wb_copy(wg - _NB, b).wait()

                @pl.when(w + 1 < wpw)
                def _():
                    gather_start(w + 1, (b + 1) % _NB)

                pltpu.make_async_copy(tok_hbm.at[idx_all.at[w]],
                                      bufs.at[b], gsem[b]).wait()

                l = wg >> 5
                pv = [pos_v[l, pl.ds(16 * h, _LANES)] for h in range(2)]

                @pl.loop(0, _W, step=4)
                def _(t0):
                    for dt in range(4):
                        col = jnp.full((_LANES,), t0 + dt, jnp.int32)
                        for h in range(2):
                            v = bufs[b, t0 + dt, pl.ds(16 * h, _LANES)]
                            plsc.store_scatter(
                                tbufs.at[b], [e_h[h], col], v + pv[h])

                wb_copy(wg, b).start()

        for b in range(_NB):
            wb_copy(base_win + wpw - _NB + b, b).wait()

    return k(x2, token_table, pos_table)


def kernel(x, token_table, pos_table):
    Bseq, L = x.shape
    x2 = jnp.transpose(x).astype(jnp.int32).reshape(-1, _W)
    tt_rm = _sc_transpose_table(jnp.transpose(token_table))
    tt = tt_rm.reshape(token_table.shape)
    p2 = _sc_embed(x2, tt, pos_table)
    p3 = p2.reshape(L, _E, Bseq)
    return jnp.transpose(p3, (2, 0, 1))


# speedup vs baseline: 4.9089x; 1.7568x over previous
import dataclasses
import functools

import jax
import jax.numpy as jnp
from jax import lax
from jax.experimental import pallas as pl
from jax.experimental.pallas import tpu as pltpu
from jax.experimental.pallas import tpu_sc as plsc

_L = 200
_E = 32
_LANES = 16
_W = 128
_NB = 4
_NWORK = 32
_TS = 129


def _compiler_params(tc_tiling=False):
    cp = pltpu.CompilerParams(use_tc_tiling_on_sc=tc_tiling)
    if "needs_layout_passes" in pltpu.CompilerParams.__dataclass_fields__:
        cp = dataclasses.replace(cp, needs_layout_passes=False)
    return cp


def _sc_transpose_table(tt):
    V = tt.shape[1]
    ngrp = V // _W
    gpw = ngrp // _NWORK
    rem = ngrp - gpw * _NWORK
    mesh = plsc.VectorSubcoreMesh(core_axis_name="core",
                                  subcore_axis_name="subcore")

    @functools.partial(
        pl.kernel,
        out_type=jax.ShapeDtypeStruct((V * _E // _W, _W), jnp.float32),
        mesh=mesh,
        compiler_params=_compiler_params(tc_tiling=True),
        scratch_types=(
            [pltpu.VMEM((_NB, _E, _W), jnp.float32),
             pltpu.VMEM((_NB, _E, _W), jnp.float32)]
            + [pltpu.SemaphoreType.DMA] * (2 * _NB)
        ),
    )
    def ka(tt_hbm, out_hbm, ibufs, obufs, *sems):
        gsem = sems[:_NB]
        wsem = sems[_NB:]
        cid = lax.axis_index("core")
        sid = lax.axis_index("subcore")
        wid = sid * 2 + cid
        base_g = wid * gpw

        iota = lax.iota(jnp.int32, _LANES)
        row_h = [iota + 16 * h for h in range(2)]

        def g_start(g, b):
            pltpu.async_copy(
                tt_hbm.at[:, pl.ds((base_g + g) * _W, _W)],
                ibufs.at[b], gsem[b])

        def g_wait(g, b):
            pltpu.make_async_copy(
                tt_hbm.at[:, pl.ds((base_g + g) * _W, _W)],
                ibufs.at[b], gsem[b]).wait()

        def wb(g, b):
            return pltpu.make_async_copy(
                obufs.at[b],
                out_hbm.at[pl.ds((base_g + g) * _E, _E)], wsem[b])

        def transpose_group(b, n_m):
            @pl.loop(0, _LANES)
            def _(k):
                permk = (iota + k) & 15
                pk32i = (permk << 5) + iota
                for h in range(2):
                    tmp = pk32i + 16 * h
                    rbase = tmp >> 7
                    cvec = tmp & 127
                    for m in range(n_m):
                        colv = permk + 16 * m
                        v = plsc.load_gather(ibufs.at[b], [row_h[h], colv])
                        plsc.store_scatter(obufs.at[b],
                                           [rbase + 4 * m, cvec], v)

        g_start(0, 0)

        @pl.loop(0, gpw, step=_NB)
        def _(g0):
            for b in range(_NB):
                g = g0 + b

                @pl.when(g + 1 < gpw)
                def _():
                    g_start(g + 1, (b + 1) % _NB)

                g_wait(g, b)

                @pl.when(g >= _NB)
                def _():
                    wb(g - _NB, b).wait()

                transpose_group(b, 8)
                wb(g, b).start()

        for b in range(_NB):
            wb(gpw - _NB + b, b).wait()

        @pl.when(wid == _NWORK - 1)
        def _():
            for k in range(rem):
                gg = _NWORK * gpw + k
                pltpu.sync_copy(tt_hbm.at[:, pl.ds(gg * _W, _W)],
                                ibufs.at[0])
                transpose_group(0, 8)
                pltpu.sync_copy(obufs.at[0],
                                out_hbm.at[pl.ds(gg * _E, _E)])
            tail = V - ngrp * _W
            off = jnp.where(wid >= 0, ngrp * _W, 0)
            pltpu.sync_copy(tt_hbm.at[:, pl.ds(off, _W)],
                            ibufs.at[0])
            transpose_group(0, tail // _LANES)
            pltpu.sync_copy(obufs.at[0, pl.ds(0, tail * _E // _W)],
                            out_hbm.at[pl.ds(ngrp * _E, tail * _E // _W)])

    return ka(tt)


def _sc_embed(x2, token_table, pos_table):
    nwin = x2.shape[0]
    wpw = nwin // _NWORK
    nbatch = x2.shape[1] * 32
    mesh = plsc.VectorSubcoreMesh(core_axis_name="core",
                                  subcore_axis_name="subcore")

    @functools.partial(
        pl.kernel,
        out_type=jax.ShapeDtypeStruct((_L * _E, nbatch), jnp.float32),
        mesh=mesh,
        compiler_params=_compiler_params(),
        scratch_types=(
            [pltpu.VMEM((wpw, _W), jnp.int32),
             pltpu.VMEM((_L, _E), jnp.float32),
             pltpu.VMEM((_NB, _W, _E), jnp.float32),
             pltpu.VMEM((_NB, _E, _TS), jnp.float32)]
            + [pltpu.SemaphoreType.DMA] * (2 * _NB)
        ),
    )
    def k(x_hbm, tok_hbm, pos_hbm, out_hbm, idx_all, pos_v,
          bufs, tbufs, *sems):
        gsem = sems[:_NB]
        wsem = sems[_NB:2 * _NB]
        cid = lax.axis_index("core")
        sid = lax.axis_index("subcore")
        wid = sid * 2 + cid
        base_win = wid * wpw

        pltpu.sync_copy(x_hbm.at[pl.ds(base_win, wpw)], idx_all)
        pltpu.sync_copy(pos_hbm, pos_v)

        iota = lax.iota(jnp.int32, _LANES)
        e_h = [iota + 16 * h for h in range(2)]

        def gather_start(w, b):
            pltpu.async_copy(tok_hbm.at[idx_all.at[w]], bufs.at[b], gsem[b])

        def wb_copy(w_glob, b):
            l = w_glob >> 5
            bt = w_glob & 31
            return pltpu.make_async_copy(
                tbufs.at[b, :, pl.ds(0, _W)],
                out_hbm.at[pl.ds(l * _E, _E), pl.ds(bt * _W, _W)],
                wsem[b])

        gather_start(0, 0)

        @pl.loop(0, wpw, step=_NB)
        def _(g):
            for b in range(_NB):
                w = g + b
                wg = base_win + w

                @pl.when(w >= _NB)
                def _():
                    wb_copy(wg - _NB, b).wait()

                @pl.when(w + 1 < wpw)
                def _():
                    gather_start(w + 1, (b + 1) % _NB)

                pltpu.make_async_copy(tok_hbm.at[idx_all.at[w]],
                                      bufs.at[b], gsem[b]).wait()

                l = wg >> 5
                pv = [pos_v[l, pl.ds(16 * h, _LANES)] for h in range(2)]

                @pl.loop(0, _W, step=4)
                def _(t0):
                    for dt in range(4):
                        col = jnp.full((_LANES,), t0 + dt, jnp.int32)
                        for h in range(2):
                            v = bufs[b, t0 + dt, pl.ds(16 * h, _LANES)]
                            plsc.store_scatter(
                                tbufs.at[b], [e_h[h], col], v + pv[h])

                wb_copy(wg, b).start()

        for b in range(_NB):
            wb_copy(base_win + wpw - _NB + b, b).wait()

    return k(x2, token_table, pos_table)


def kernel(x, token_table, pos_table):
    Bseq, L = x.shape
    x2 = jnp.transpose(x).astype(jnp.int32).reshape(-1, _W)
    tt_rm = _sc_transpose_table(jnp.transpose(token_table))
    tt = tt_rm.reshape(token_table.shape)
    p2 = _sc_embed(x2, tt, pos_table)
    p3 = p2.reshape(L, _E, Bseq)
    return jnp.transpose(p3, (2, 0, 1))
